# baseline (speedup 1.0000x reference)
import functools

import jax
import jax.numpy as jnp
from jax import lax
from jax.experimental import pallas as pl
from jax.experimental.pallas import tpu as pltpu
from jax.experimental.pallas import tpu_sc as plsc

N = 8192
K = 16
D = 64
BLK = 256
NBLK = N // BLK
SLOPE = 0.2
E = N * K
GCHUNK = 128

_BF = jnp.bfloat16


def _leaky(h):
    return jnp.where(h >= 0, h, SLOPE * h)


def _mm(a, b):
    return jnp.dot(a.astype(_BF), b.astype(_BF),
                   preferred_element_type=jnp.float32)


def _knn_body(x_ref, xt_ref, idx_ref):
    xb = x_ref[...]
    xt = xt_ref[...]
    sqj = jnp.sum(xt * xt, axis=0, keepdims=True)
    sqi = jnp.sum(xb * xb, axis=1, keepdims=True)
    d = (sqi - 2.0 * _mm(xb, xt)) + sqj

    NCH = 16
    W = N // NCH
    big = jnp.float32(3.0e38)
    tiles = [d[:, c * W:(c + 1) * W] for c in range(NCH)]
    mins = [jnp.min(t, axis=1, keepdims=True) for t in tiles]
    R = jnp.concatenate(mins, axis=1)
    chunkiota = lax.broadcasted_iota(jnp.int32, (BLK, NCH), 1)
    laneiota = lax.broadcasted_iota(jnp.int32, (BLK, W), 1)
    cols = []
    for _ in range(K):
        m = jnp.min(R, axis=1, keepdims=True)
        cstar = jnp.min(jnp.where(R == m, chunkiota, NCH),
                        axis=1, keepdims=True)
        t = tiles[0]
        for c in range(1, NCH):
            t = jnp.where(cstar == c, tiles[c], t)
        jl = jnp.min(jnp.where(t == m, laneiota, W),
                     axis=1, keepdims=True)
        cols.append(cstar * W + jl)
        tn = jnp.where(laneiota == jl, big, t)
        for c in range(NCH):
            tiles[c] = jnp.where(cstar == c, tn, tiles[c])
        mnew = jnp.min(tn, axis=1, keepdims=True)
        R = jnp.where(chunkiota == cstar, mnew, R)
    idx_ref[...] = jnp.concatenate(cols, axis=1)


def _knn_call(x, xt):
    return pl.pallas_call(
        _knn_body,
        grid=(NBLK,),
        in_specs=[
            pl.BlockSpec((BLK, D), lambda i: (i, 0)),
            pl.BlockSpec((D, N), lambda i: (0, 0)),
        ],
        out_specs=pl.BlockSpec((BLK, K), lambda i: (i, 0)),
        out_shape=jax.ShapeDtypeStruct((N, K), jnp.int32),
        compiler_params=pltpu.CompilerParams(
            dimension_semantics=("arbitrary",)),
    )(x, xt)


def _sc_gather(table, idx_flat):
    info = plsc.get_sparse_core_info()
    nc, ns = info.num_cores, info.num_subcores
    nw = nc * ns
    e_per_w = E // nw
    nch = e_per_w // GCHUNK

    mesh = plsc.VectorSubcoreMesh(core_axis_name="c", subcore_axis_name="s")

    @functools.partial(
        pl.kernel, mesh=mesh,
        out_type=jax.ShapeDtypeStruct((E, D), jnp.float32),
        scratch_types=[
            pltpu.VMEM((GCHUNK,), jnp.int32),
            pltpu.VMEM((GCHUNK, D), jnp.float32),
            pltpu.SemaphoreType.DMA,
        ],
        compiler_params=pltpu.CompilerParams(use_tc_tiling_on_sc=False),
    )
    def gk(table_hbm, idx_hbm, out_hbm, idx_v, rows_v, sem):
        wid = lax.axis_index("s") * nc + lax.axis_index("c")
        base = wid * e_per_w

        def body(c, _):
            off = base + c * GCHUNK
            pltpu.sync_copy(idx_hbm.at[pl.ds(off, GCHUNK)], idx_v)
            pltpu.async_copy(table_hbm.at[idx_v], rows_v, sem).wait()
            pltpu.sync_copy(rows_v, out_hbm.at[pl.ds(off, GCHUNK)])
            return 0

        lax.fori_loop(0, nch, body, 0)

    return gk(table, idx_flat)


def _mlp_body(x_ref, xg_ref, wtop_ref, wbot_ref, ba_ref,
              wb_ref, bb_ref, wc_ref, bc_ref, o_ref):
    xi = x_ref[...]
    wtop = wtop_ref[...]
    wbot = wbot_ref[...]
    wb = wb_ref[...]
    wc = wc_ref[...]
    ba = ba_ref[...]
    bb = bb_ref[...]
    bc = bc_ref[...]
    base = _mm(xi, wtop)
    acc = jnp.full((BLK, D), -jnp.inf, jnp.float32)
    for k in range(K):
        t = xg_ref[k] - xi
        h1 = _leaky(base + _mm(t, wbot) + ba)
        h2 = _leaky(_mm(h1, wb) + bb)
        h3 = _leaky(_mm(h2, wc) + bc)
        acc = jnp.maximum(acc, h3)
    o_ref[...] = acc


def _mlp_call(x, xg, wtop, wbot, ba, wb, bb, wc, bc):
    return pl.pallas_call(
        _mlp_body,
        grid=(NBLK,),
        in_specs=[
            pl.BlockSpec((BLK, D), lambda i: (i, 0)),
            pl.BlockSpec((K, BLK, D), lambda i: (0, i, 0)),
            pl.BlockSpec((D, D), lambda i: (0, 0)),
            pl.BlockSpec((D, D), lambda i: (0, 0)),
            pl.BlockSpec((1, D), lambda i: (0, 0)),
            pl.BlockSpec((D, D), lambda i: (0, 0)),
            pl.BlockSpec((1, D), lambda i: (0, 0)),
            pl.BlockSpec((D, D), lambda i: (0, 0)),
            pl.BlockSpec((1, D), lambda i: (0, 0)),
        ],
        out_specs=pl.BlockSpec((BLK, D), lambda i: (i, 0)),
        out_shape=jax.ShapeDtypeStruct((N, D), jnp.float32),
        compiler_params=pltpu.CompilerParams(
            dimension_semantics=("arbitrary",)),
    )(x, xg, wtop, wbot, ba, wb, bb, wc, bc)


def _edge_conv_layer(x, wtop, wbot, ba, wb, bb, wc, bc):
    idx = _knn_call(x, x.T)
    idx_flat = idx.T.reshape(E)
    xg = _sc_gather(x, idx_flat).reshape(K, N, D)
    return _mlp_call(x, xg, wtop, wbot, ba.reshape(1, D),
                     wb, bb.reshape(1, D), wc, bc.reshape(1, D))


def _pad_half(w, din):
    return jnp.zeros((D, D), jnp.float32).at[:din].set(w)


def kernel(pos, W1a, b1a, W1b, b1b, W1c, b1c,
           W2a, b2a, W2b, b2b, W2c, b2c,
           W3a, b3a, W3b, b3b, W3c, b3c):
    x = jnp.zeros((N, D), jnp.float32).at[:, :pos.shape[1]].set(pos)
    x = _edge_conv_layer(x, _pad_half(W1a[:3], 3), _pad_half(W1a[3:], 3),
                         b1a, W1b, b1b, W1c, b1c)
    x = _edge_conv_layer(x, W2a[:D], W2a[D:], b2a, W2b, b2b, W2c, b2c)
    x = _edge_conv_layer(x, W3a[:D], W3a[D:], b3a, W3b, b3b, W3c, b3c)
    return x


# speedup vs baseline: 1.0122x; 1.0122x over previous
import functools

import jax
import jax.numpy as jnp
from jax import lax
from jax.experimental import pallas as pl
from jax.experimental.pallas import tpu as pltpu
from jax.experimental.pallas import tpu_sc as plsc

N = 8192
K = 16
D = 64
BLK = 256
NBLK = N // BLK
SLOPE = 0.2
E = N * K
GCHUNK = 128

_BF = jnp.bfloat16


def _leaky(h):
    return jnp.where(h >= 0, h, SLOPE * h)


def _mm(a, b):
    return jnp.dot(a.astype(_BF), b.astype(_BF),
                   preferred_element_type=jnp.float32)


def _knn_body(x_ref, xt_ref, idx_ref):
    xb = x_ref[...]
    xt = xt_ref[...]
    sqj = jnp.sum(xt * xt, axis=0, keepdims=True)
    sqi = jnp.sum(xb * xb, axis=1, keepdims=True)
    d = (sqi - 2.0 * _mm(xb, xt)) + sqj

    NCH = 64
    W = N // NCH
    big = jnp.float32(3.0e38)
    tiles = [d[:, c * W:(c + 1) * W] for c in range(NCH)]
    mins = [jnp.min(t, axis=1, keepdims=True) for t in tiles]
    R = jnp.concatenate(mins, axis=1)
    chunkiota = lax.broadcasted_iota(jnp.int32, (BLK, NCH), 1)
    laneiota = lax.broadcasted_iota(jnp.int32, (BLK, W), 1)
    cols = []
    for _ in range(K):
        m = jnp.min(R, axis=1, keepdims=True)
        cstar = jnp.min(jnp.where(R == m, chunkiota, NCH),
                        axis=1, keepdims=True)
        t = tiles[0]
        for c in range(1, NCH):
            t = jnp.where(cstar == c, tiles[c], t)
        jl = jnp.min(jnp.where(t == m, laneiota, W),
                     axis=1, keepdims=True)
        cols.append(cstar * W + jl)
        tn = jnp.where(laneiota == jl, big, t)
        for c in range(NCH):
            tiles[c] = jnp.where(cstar == c, tn, tiles[c])
        mnew = jnp.min(tn, axis=1, keepdims=True)
        R = jnp.where(chunkiota == cstar, mnew, R)
    idx_ref[...] = jnp.concatenate(cols, axis=1)


def _knn_call(x, xt):
    return pl.pallas_call(
        _knn_body,
        grid=(NBLK,),
        in_specs=[
            pl.BlockSpec((BLK, D), lambda i: (i, 0)),
            pl.BlockSpec((D, N), lambda i: (0, 0)),
        ],
        out_specs=pl.BlockSpec((BLK, K), lambda i: (i, 0)),
        out_shape=jax.ShapeDtypeStruct((N, K), jnp.int32),
        compiler_params=pltpu.CompilerParams(
            dimension_semantics=("arbitrary",)),
    )(x, xt)


def _sc_gather(table, idx_flat):
    info = plsc.get_sparse_core_info()
    nc, ns = info.num_cores, info.num_subcores
    nw = nc * ns
    e_per_w = E // nw
    nch = e_per_w // GCHUNK

    mesh = plsc.VectorSubcoreMesh(core_axis_name="c", subcore_axis_name="s")

    @functools.partial(
        pl.kernel, mesh=mesh,
        out_type=jax.ShapeDtypeStruct((E, D), jnp.float32),
        scratch_types=[
            pltpu.VMEM((GCHUNK,), jnp.int32),
            pltpu.VMEM((GCHUNK, D), jnp.float32),
            pltpu.SemaphoreType.DMA,
        ],
        compiler_params=pltpu.CompilerParams(use_tc_tiling_on_sc=False),
    )
    def gk(table_hbm, idx_hbm, out_hbm, idx_v, rows_v, sem):
        wid = lax.axis_index("s") * nc + lax.axis_index("c")
        base = wid * e_per_w

        def body(c, _):
            off = base + c * GCHUNK
            pltpu.sync_copy(idx_hbm.at[pl.ds(off, GCHUNK)], idx_v)
            pltpu.async_copy(table_hbm.at[idx_v], rows_v, sem).wait()
            pltpu.sync_copy(rows_v, out_hbm.at[pl.ds(off, GCHUNK)])
            return 0

        lax.fori_loop(0, nch, body, 0)

    return gk(table, idx_flat)


def _mlp_body(x_ref, xg_ref, wtop_ref, wbot_ref, ba_ref,
              wb_ref, bb_ref, wc_ref, bc_ref, o_ref):
    xi = x_ref[...]
    wtop = wtop_ref[...]
    wbot = wbot_ref[...]
    wb = wb_ref[...]
    wc = wc_ref[...]
    ba = ba_ref[...]
    bb = bb_ref[...]
    bc = bc_ref[...]
    base = _mm(xi, wtop)
    acc = jnp.full((BLK, D), -jnp.inf, jnp.float32)
    for k in range(K):
        t = xg_ref[k] - xi
        h1 = _leaky(base + _mm(t, wbot) + ba)
        h2 = _leaky(_mm(h1, wb) + bb)
        h3 = _leaky(_mm(h2, wc) + bc)
        acc = jnp.maximum(acc, h3)
    o_ref[...] = acc


def _mlp_call(x, xg, wtop, wbot, ba, wb, bb, wc, bc):
    return pl.pallas_call(
        _mlp_body,
        grid=(NBLK,),
        in_specs=[
            pl.BlockSpec((BLK, D), lambda i: (i, 0)),
            pl.BlockSpec((K, BLK, D), lambda i: (0, i, 0)),
            pl.BlockSpec((D, D), lambda i: (0, 0)),
            pl.BlockSpec((D, D), lambda i: (0, 0)),
            pl.BlockSpec((1, D), lambda i: (0, 0)),
            pl.BlockSpec((D, D), lambda i: (0, 0)),
            pl.BlockSpec((1, D), lambda i: (0, 0)),
            pl.BlockSpec((D, D), lambda i: (0, 0)),
            pl.BlockSpec((1, D), lambda i: (0, 0)),
        ],
        out_specs=pl.BlockSpec((BLK, D), lambda i: (i, 0)),
        out_shape=jax.ShapeDtypeStruct((N, D), jnp.float32),
        compiler_params=pltpu.CompilerParams(
            dimension_semantics=("arbitrary",)),
    )(x, xg, wtop, wbot, ba, wb, bb, wc, bc)


def _edge_conv_layer(x, wtop, wbot, ba, wb, bb, wc, bc):
    idx = _knn_call(x, x.T)
    idx_flat = idx.T.reshape(E)
    xg = _sc_gather(x, idx_flat).reshape(K, N, D)
    return _mlp_call(x, xg, wtop, wbot, ba.reshape(1, D),
                     wb, bb.reshape(1, D), wc, bc.reshape(1, D))


def _pad_half(w, din):
    return jnp.zeros((D, D), jnp.float32).at[:din].set(w)


def kernel(pos, W1a, b1a, W1b, b1b, W1c, b1c,
           W2a, b2a, W2b, b2b, W2c, b2c,
           W3a, b3a, W3b, b3b, W3c, b3c):
    x = jnp.zeros((N, D), jnp.float32).at[:, :pos.shape[1]].set(pos)
    x = _edge_conv_layer(x, _pad_half(W1a[:3], 3), _pad_half(W1a[3:], 3),
                         b1a, W1b, b1b, W1c, b1c)
    x = _edge_conv_layer(x, W2a[:D], W2a[D:], b2a, W2b, b2b, W2c, b2c)
    x = _edge_conv_layer(x, W3a[:D], W3a[D:], b3a, W3b, b3b, W3c, b3c)
    return x


# speedup vs baseline: 1.1244x; 1.1109x over previous
import functools

import jax
import jax.numpy as jnp
from jax import lax
from jax.experimental import pallas as pl
from jax.experimental.pallas import tpu as pltpu
from jax.experimental.pallas import tpu_sc as plsc

N = 8192
K = 16
D = 64
BLK = 256
NBLK = N // BLK
SLOPE = 0.2
E = N * K
GCHUNK = 128

_BF = jnp.bfloat16


def _leaky(h):
    return jnp.where(h >= 0, h, SLOPE * h)


def _mm(a, b):
    return jnp.dot(a.astype(_BF), b.astype(_BF),
                   preferred_element_type=jnp.float32)


def _knn_body(x_ref, xt_ref, idx_ref):
    xb = x_ref[...]
    xt = xt_ref[...]
    sqj = jnp.sum(xt * xt, axis=0, keepdims=True)
    sqi = jnp.sum(xb * xb, axis=1, keepdims=True)
    d = (sqi - 2.0 * _mm(xb, xt)) + sqj

    NCH = 32
    W = N // NCH
    big = jnp.float32(3.0e38)
    tiles = [d[:, c * W:(c + 1) * W] for c in range(NCH)]
    mins = [jnp.min(t, axis=1, keepdims=True) for t in tiles]
    R = jnp.concatenate(mins, axis=1)
    chunkiota = lax.broadcasted_iota(jnp.int32, (BLK, NCH), 1)
    laneiota = lax.broadcasted_iota(jnp.int32, (BLK, W), 1)
    cols = []
    for _ in range(K):
        m = jnp.min(R, axis=1, keepdims=True)
        cstar = jnp.min(jnp.where(R == m, chunkiota, NCH),
                        axis=1, keepdims=True)
        t = tiles[0]
        for c in range(1, NCH):
            t = jnp.where(cstar == c, tiles[c], t)
        jl = jnp.min(jnp.where(t == m, laneiota, W),
                     axis=1, keepdims=True)
        cols.append(cstar * W + jl)
        tn = jnp.where(laneiota == jl, big, t)
        for c in range(NCH):
            tiles[c] = jnp.where(cstar == c, tn, tiles[c])
        mnew = jnp.min(tn, axis=1, keepdims=True)
        R = jnp.where(chunkiota == cstar, mnew, R)
    idx_ref[...] = jnp.concatenate(cols, axis=1)


def _knn_call(x, xt):
    return pl.pallas_call(
        _knn_body,
        grid=(NBLK,),
        in_specs=[
            pl.BlockSpec((BLK, D), lambda i: (i, 0)),
            pl.BlockSpec((D, N), lambda i: (0, 0)),
        ],
        out_specs=pl.BlockSpec((BLK, K), lambda i: (i, 0)),
        out_shape=jax.ShapeDtypeStruct((N, K), jnp.int32),
        compiler_params=pltpu.CompilerParams(
            dimension_semantics=("arbitrary",)),
    )(x, xt)


def _sc_gather(table, idx_flat):
    info = plsc.get_sparse_core_info()
    nc, ns = info.num_cores, info.num_subcores
    nw = nc * ns
    e_per_w = E // nw
    nch = e_per_w // GCHUNK

    mesh = plsc.VectorSubcoreMesh(core_axis_name="c", subcore_axis_name="s")

    nbuf = 4

    @functools.partial(
        pl.kernel, mesh=mesh,
        out_type=jax.ShapeDtypeStruct((E, D), jnp.float32),
        scratch_types=[
            pltpu.VMEM((nch, GCHUNK), jnp.int32),
            pltpu.VMEM((GCHUNK, D), jnp.float32),
            pltpu.VMEM((GCHUNK, D), jnp.float32),
            pltpu.VMEM((GCHUNK, D), jnp.float32),
            pltpu.VMEM((GCHUNK, D), jnp.float32),
            pltpu.SemaphoreType.DMA,
            pltpu.SemaphoreType.DMA,
            pltpu.SemaphoreType.DMA,
            pltpu.SemaphoreType.DMA,
        ],
        compiler_params=pltpu.CompilerParams(use_tc_tiling_on_sc=False),
    )
    def gk(table_hbm, idx_hbm, out_hbm, idx_v, r0, r1, r2, r3,
           s0, s1, s2, s3):
        wid = lax.axis_index("s") * nc + lax.axis_index("c")
        base = wid * e_per_w
        pltpu.sync_copy(idx_hbm.at[pl.ds(wid * nch, nch)], idx_v)
        rows = [r0, r1, r2, r3]
        sems = [s0, s1, s2, s3]

        def body(g, _):
            j = g * nbuf
            cps = [
                pltpu.async_copy(table_hbm.at[idx_v.at[j + b]],
                                 rows[b], sems[b])
                for b in range(nbuf)
            ]
            for b in range(nbuf):
                cps[b].wait()
                pltpu.sync_copy(
                    rows[b],
                    out_hbm.at[pl.ds(base + (j + b) * GCHUNK, GCHUNK)])
            return 0

        lax.fori_loop(0, nch // nbuf, body, 0)

    return gk(table, idx_flat.reshape(nw * nch, GCHUNK))


def _mlp_body(x_ref, xg_ref, wtop_ref, wbot_ref, ba_ref,
              wb_ref, bb_ref, wc_ref, bc_ref, o_ref):
    xi = x_ref[...]
    wtop = wtop_ref[...]
    wbot = wbot_ref[...]
    wb = wb_ref[...]
    wc = wc_ref[...]
    ba = ba_ref[...]
    bb = bb_ref[...]
    bc = bc_ref[...]
    base = _mm(xi, wtop)
    acc = jnp.full((BLK, D), -jnp.inf, jnp.float32)
    for k in range(K):
        t = xg_ref[k] - xi
        h1 = _leaky(base + _mm(t, wbot) + ba)
        h2 = _leaky(_mm(h1, wb) + bb)
        h3 = _leaky(_mm(h2, wc) + bc)
        acc = jnp.maximum(acc, h3)
    o_ref[...] = acc


def _mlp_call(x, xg, wtop, wbot, ba, wb, bb, wc, bc):
    return pl.pallas_call(
        _mlp_body,
        grid=(NBLK,),
        in_specs=[
            pl.BlockSpec((BLK, D), lambda i: (i, 0)),
            pl.BlockSpec((K, BLK, D), lambda i: (0, i, 0)),
            pl.BlockSpec((D, D), lambda i: (0, 0)),
            pl.BlockSpec((D, D), lambda i: (0, 0)),
            pl.BlockSpec((1, D), lambda i: (0, 0)),
            pl.BlockSpec((D, D), lambda i: (0, 0)),
            pl.BlockSpec((1, D), lambda i: (0, 0)),
            pl.BlockSpec((D, D), lambda i: (0, 0)),
            pl.BlockSpec((1, D), lambda i: (0, 0)),
        ],
        out_specs=pl.BlockSpec((BLK, D), lambda i: (i, 0)),
        out_shape=jax.ShapeDtypeStruct((N, D), jnp.float32),
        compiler_params=pltpu.CompilerParams(
            dimension_semantics=("arbitrary",)),
    )(x, xg, wtop, wbot, ba, wb, bb, wc, bc)


def _edge_conv_layer(x, wtop, wbot, ba, wb, bb, wc, bc):
    idx = _knn_call(x, x.T)
    idx_flat = idx.T.reshape(E)
    xg = _sc_gather(x, idx_flat).reshape(K, N, D)
    return _mlp_call(x, xg, wtop, wbot, ba.reshape(1, D),
                     wb, bb.reshape(1, D), wc, bc.reshape(1, D))


def _pad_half(w, din):
    return jnp.zeros((D, D), jnp.float32).at[:din].set(w)


def kernel(pos, W1a, b1a, W1b, b1b, W1c, b1c,
           W2a, b2a, W2b, b2b, W2c, b2c,
           W3a, b3a, W3b, b3b, W3c, b3c):
    x = jnp.zeros((N, D), jnp.float32).at[:, :pos.shape[1]].set(pos)
    x = _edge_conv_layer(x, _pad_half(W1a[:3], 3), _pad_half(W1a[3:], 3),
                         b1a, W1b, b1b, W1c, b1c)
    x = _edge_conv_layer(x, W2a[:D], W2a[D:], b2a, W2b, b2b, W2c, b2c)
    x = _edge_conv_layer(x, W3a[:D], W3a[D:], b3a, W3b, b3b, W3c, b3c)
    return x
